# baseline (speedup 1.0000x reference)
import functools

import jax
import jax.numpy as jnp
from jax.experimental import pallas as pl
from jax.experimental.pallas import tpu as pltpu

_NC = 128


def _chamfer_kernel(c1, c2, pred_ref, tgt_ref, loss_ref):
    b = pl.program_id(0)
    p = pred_ref[0]
    t = tgt_ref[0]
    N = p.shape[0]
    M = t.shape[1]

    @pl.when(b == 0)
    def _():
        loss_ref[...] = jnp.zeros((1, 1), jnp.float32)

    p2 = jnp.sum(p * p, axis=1, keepdims=True)
    p2h = p2.astype(jnp.bfloat16).astype(jnp.float32)
    p2l = p2 - p2h
    ones_n = jnp.ones((N, 1), jnp.float32)
    zero_n = jnp.zeros((N, 1), jnp.float32)
    lhs = jnp.concatenate(
        [jnp.swapaxes(-2.0 * p, 0, 1), jnp.swapaxes(p2h, 0, 1),
         jnp.swapaxes(p2l, 0, 1), jnp.ones((3, N), jnp.float32)[:2],
         jnp.zeros((1, N), jnp.float32)],
        axis=0).astype(jnp.bfloat16)

    t2 = jnp.sum(t * t, axis=0, keepdims=True)
    t2h = t2.astype(jnp.bfloat16).astype(jnp.float32)
    t2l = t2 - t2h
    ones_m = jnp.ones((1, M), jnp.float32)
    zero_m = jnp.zeros((1, M), jnp.float32)
    rhs = jnp.concatenate(
        [t, ones_m, ones_m, t2h, t2l, zero_m],
        axis=0).astype(jnp.bfloat16)

    row_sum = None
    colmin8 = None
    for c in range(N // _NC):
        f = jax.lax.dot_general(
            lhs[:, c * _NC:(c + 1) * _NC], rhs, (((0,), (0,)), ((), ())),
            preferred_element_type=jnp.float32)
        rowmin = jnp.min(f, axis=1, keepdims=True)
        rs = jnp.sum(jnp.maximum(rowmin, 0.0), axis=0, keepdims=True)
        row_sum = rs if row_sum is None else row_sum + rs
        cm8 = jnp.min(f.reshape(_NC // 8, 8, M), axis=0)
        colmin8 = cm8 if colmin8 is None else jnp.minimum(colmin8, cm8)

    colmin = jnp.min(jnp.maximum(colmin8, 0.0), axis=0, keepdims=True)
    col_sum = jnp.sum(colmin, axis=1, keepdims=True)
    loss_ref[...] += row_sum * c1 + col_sum * c2


def kernel(pred, target):
    B, N, D = pred.shape
    M = target.shape[1]
    tgt = jnp.swapaxes(target, 1, 2)
    c1 = 0.5 / (B * N)
    c2 = 0.5 / (B * M)
    loss = pl.pallas_call(
        functools.partial(_chamfer_kernel, c1, c2),
        grid=(B,),
        in_specs=[
            pl.BlockSpec((1, N, D), lambda b: (b, 0, 0)),
            pl.BlockSpec((1, D, M), lambda b: (b, 0, 0)),
        ],
        out_specs=pl.BlockSpec((1, 1), lambda b: (0, 0)),
        out_shape=jax.ShapeDtypeStruct((1, 1), jnp.float32),
    )(pred, tgt)
    return loss[0, 0]
